# baseline (device time: 183704 ns/iter reference)
import jax
import jax.numpy as jnp
from jax import lax
from jax.experimental import pallas as pl
from jax.experimental.pallas import tpu as pltpu

N_DEV = 16


def kernel(x, w_mat):
    m_per, k = x.shape
    _, n_per = w_mat.shape

    def body(x_ref, w_ref, out_ref, comm_ref, w_bf_ref,
             ss_fo, rs_fo, ss_fp, rs_fp, ss_bo, rs_bo, ss_bp, rs_bp,
             ss_x, rs_x):
        my_pos = lax.axis_index("i")
        z = my_pos // 4
        i = lax.rem(my_pos, 4)
        xc = jnp.where((i == 1) | (i == 2), 1, 0)
        yc = jnp.where(i >= 2, 1, 0)
        i0 = xc
        i1 = 3 - xc
        j = jnp.where(yc == 0, z, 7 - z)

        def ring_pos(jj):
            jj = lax.rem(jj + 16, 8)
            return jnp.where(jj < 4, 4 * jj + i0, 4 * (7 - jj) + i1)

        fwd = ring_pos(j + 1)
        bwd = ring_pos(j - 1)
        partner = 4 * z + jnp.bitwise_xor(i, 1)

        barrier_sem = pltpu.get_barrier_semaphore()
        for nbr in [fwd, bwd, partner]:
            pl.semaphore_signal(
                barrier_sem, inc=1,
                device_id=(nbr,), device_id_type=pl.DeviceIdType.MESH,
            )
        comm_ref[0] = x_ref[...].astype(jnp.bfloat16)
        w_bf_ref[...] = w_ref[...].astype(jnp.bfloat16)
        pl.semaphore_wait(barrier_sem, 3)

        sends = []

        def start(src, dst, ss, rs, idx, target):
            rd = pltpu.make_async_remote_copy(
                src_ref=comm_ref.at[src],
                dst_ref=comm_ref.at[dst],
                send_sem=ss.at[idx],
                recv_sem=rs.at[idx],
                device_id=(target,),
                device_id_type=pl.DeviceIdType.MESH,
            )
            rd.start()
            sends.append(rd)
            return rd

        def gemm(slot, origin):
            out_ref[pl.ds(origin * m_per, m_per), :] = jnp.dot(
                comm_ref[slot], w_bf_ref[...],
                preferred_element_type=jnp.float32,
            )

        x_rd = start(0, 1, ss_x, rs_x, 0, partner)
        fo = {1: start(0, 2, ss_fo, rs_fo, 0, fwd)}
        bo = {1: start(0, 9, ss_bo, rs_bo, 0, bwd)}
        gemm(0, my_pos)

        x_rd.wait_recv()
        fp = {1: start(1, 3, ss_fp, rs_fp, 0, fwd)}
        bp = {1: start(1, 10, ss_bp, rs_bp, 0, bwd)}
        gemm(1, partner)

        for h in range(2, 5):
            fo[h - 1].wait_recv()
            fo[h] = start(2 * (h - 1), 2 * h, ss_fo, rs_fo, h - 1, fwd)
            bo[h - 1].wait_recv()
            if h <= 3:
                bo[h] = start(2 * (h - 1) + 7, 2 * h + 7,
                              ss_bo, rs_bo, h - 1, bwd)
            fp[h - 1].wait_recv()
            if h <= 3:
                fp[h] = start(2 * h - 1, 2 * h + 1,
                              ss_fp, rs_fp, h - 1, fwd)
            bp[h - 1].wait_recv()
            bp[h] = start(2 * (h - 1) + 8, 15 if h == 4 else 2 * h + 8,
                          ss_bp, rs_bp, h - 1, bwd)

            qf = ring_pos(j - (h - 1))
            qb = ring_pos(j + (h - 1))
            gemm(2 * (h - 1), qf)
            gemm(2 * (h - 1) + 1, jnp.bitwise_xor(qf, 1))
            gemm(2 * (h - 1) + 7, qb)
            gemm(2 * (h - 1) + 8, jnp.bitwise_xor(qb, 1))

        fo[4].wait_recv()
        bp[4].wait_recv()
        q4 = ring_pos(j + 4)
        gemm(8, q4)
        gemm(15, jnp.bitwise_xor(q4, 1))

        for rd in sends:
            rd.wait_send()

    return pl.pallas_call(
        body,
        out_shape=jax.ShapeDtypeStruct((N_DEV * m_per, n_per), jnp.float32),
        in_specs=[
            pl.BlockSpec(memory_space=pltpu.VMEM),
            pl.BlockSpec(memory_space=pltpu.VMEM),
        ],
        out_specs=pl.BlockSpec(memory_space=pltpu.VMEM),
        scratch_shapes=[
            pltpu.VMEM((N_DEV, m_per, k), jnp.bfloat16),
            pltpu.VMEM((k, n_per), jnp.bfloat16),
            pltpu.SemaphoreType.DMA((4,)),
            pltpu.SemaphoreType.DMA((4,)),
            pltpu.SemaphoreType.DMA((3,)),
            pltpu.SemaphoreType.DMA((3,)),
            pltpu.SemaphoreType.DMA((3,)),
            pltpu.SemaphoreType.DMA((3,)),
            pltpu.SemaphoreType.DMA((4,)),
            pltpu.SemaphoreType.DMA((4,)),
            pltpu.SemaphoreType.DMA((1,)),
            pltpu.SemaphoreType.DMA((1,)),
        ],
        compiler_params=pltpu.CompilerParams(
            collective_id=0,
            vmem_limit_bytes=62 * 1024 * 1024,
        ),
    )(x, w_mat)


# device time: 147511 ns/iter; 1.2454x vs baseline; 1.2454x over previous
import jax
import jax.numpy as jnp
from jax import lax
from jax.experimental import pallas as pl
from jax.experimental.pallas import tpu as pltpu

N_DEV = 16


def kernel(x, w_mat):
    x = x.astype(jnp.bfloat16)
    w_mat = w_mat.astype(jnp.bfloat16)
    m_per, k = x.shape
    _, n_per = w_mat.shape
    mh = m_per // 2

    def body(x_ref, w_ref, out_ref, comm_ref,
             ss_f, rs_f, ss_b, rs_b, ss_x, rs_x):
        my_pos = lax.axis_index("i")
        z = my_pos // 4
        i = lax.rem(my_pos, 4)
        xc = jnp.where((i == 1) | (i == 2), 1, 0)
        yc = jnp.where(i >= 2, 1, 0)
        i0 = xc
        i1 = 3 - xc
        j = jnp.where(yc == 0, z, 7 - z)

        def ring_pos(jj):
            jj = lax.rem(jj + 16, 8)
            return jnp.where(jj < 4, 4 * jj + i0, 4 * (7 - jj) + i1)

        def mate(q):
            return jnp.bitwise_xor(q, 1)

        fwd = ring_pos(j + 1)
        bwd = ring_pos(j - 1)
        partner = mate(my_pos)

        A = pl.ds(0, mh)
        B = pl.ds(mh, mh)

        barrier_sem = pltpu.get_barrier_semaphore()
        for nbr in [fwd, bwd, partner]:
            pl.semaphore_signal(
                barrier_sem, inc=1,
                device_id=(nbr,), device_id_type=pl.DeviceIdType.MESH,
            )
        comm_ref[0] = x_ref[...]
        pl.semaphore_wait(barrier_sem, 3)

        sends = []

        def start(src, dst, ss, rs, idx, target):
            src_ref = comm_ref.at[src] if isinstance(src, int) \
                else comm_ref.at[src[0], src[1]]
            dst_ref = comm_ref.at[dst] if isinstance(dst, int) \
                else comm_ref.at[dst[0], dst[1]]
            rd = pltpu.make_async_remote_copy(
                src_ref=src_ref, dst_ref=dst_ref,
                send_sem=ss.at[idx], recv_sem=rs.at[idx],
                device_id=(target,), device_id_type=pl.DeviceIdType.MESH,
            )
            rd.start()
            sends.append(rd)
            return rd

        def gemm(slot, origin):
            out_ref[pl.ds(origin * m_per, m_per), :] = jnp.dot(
                comm_ref[slot], w_ref[...],
                preferred_element_type=jnp.float32,
            )

        x0 = start(0, 1, ss_x, rs_x, 0, partner)
        f1 = start(0, 2, ss_f, rs_f, 0, fwd)
        b1 = start(0, 8, ss_b, rs_b, 0, bwd)
        gemm(0, my_pos)

        x0.wait_recv()
        gemm(1, partner)

        f1.wait_recv()
        f2 = start(2, 3, ss_f, rs_f, 1, fwd)
        x1 = start(2, 12, ss_x, rs_x, 1, partner)
        gemm(2, ring_pos(j - 1))

        b1.wait_recv()
        b2 = start(8, 9, ss_b, rs_b, 1, bwd)
        gemm(8, ring_pos(j + 1))

        f2.wait_recv()
        f3 = start(3, 4, ss_f, rs_f, 2, fwd)
        x3 = start(3, 14, ss_x, rs_x, 2, partner)
        gemm(3, ring_pos(j - 2))

        b2.wait_recv()
        b3 = start(9, 10, ss_b, rs_b, 2, bwd)
        x4 = start(9, 15, ss_x, rs_x, 3, partner)
        x2 = start(8, 13, ss_x, rs_x, 4, partner)
        gemm(9, ring_pos(j + 2))

        f3.wait_recv()
        x3.wait_recv()
        f4 = start(14, 5, ss_f, rs_f, 3, fwd)
        f5 = start((4, A), (6, A), ss_f, rs_f, 4, fwd)
        gemm(4, ring_pos(j - 3))
        gemm(14, mate(ring_pos(j - 2)))

        b3.wait_recv()
        b5 = start((10, B), (6, B), ss_b, rs_b, 3, bwd)
        gemm(10, ring_pos(j + 3))

        x1.wait_recv()
        gemm(12, mate(ring_pos(j - 1)))

        x4.wait_recv()
        b4 = start(15, 11, ss_b, rs_b, 4, bwd)
        gemm(15, mate(ring_pos(j + 2)))

        f4.wait_recv()
        f6 = start((5, A), (7, A), ss_f, rs_f, 5, fwd)
        gemm(5, mate(ring_pos(j - 3)))

        b5.wait_recv()
        f5.wait_recv()
        q4 = ring_pos(j + 4)
        gemm(6, q4)

        x2.wait_recv()
        gemm(13, mate(ring_pos(j + 1)))

        b4.wait_recv()
        b6 = start((11, B), (7, B), ss_b, rs_b, 5, bwd)
        gemm(11, mate(ring_pos(j + 3)))

        f6.wait_recv()
        b6.wait_recv()
        gemm(7, mate(q4))

        for rd in sends:
            rd.wait_send()

    return pl.pallas_call(
        body,
        out_shape=jax.ShapeDtypeStruct((N_DEV * m_per, n_per), jnp.float32),
        in_specs=[
            pl.BlockSpec(memory_space=pltpu.VMEM),
            pl.BlockSpec(memory_space=pltpu.VMEM),
        ],
        out_specs=pl.BlockSpec(memory_space=pltpu.VMEM),
        scratch_shapes=[
            pltpu.VMEM((N_DEV, m_per, k), jnp.bfloat16),
            pltpu.SemaphoreType.DMA((6,)),
            pltpu.SemaphoreType.DMA((6,)),
            pltpu.SemaphoreType.DMA((6,)),
            pltpu.SemaphoreType.DMA((6,)),
            pltpu.SemaphoreType.DMA((5,)),
            pltpu.SemaphoreType.DMA((5,)),
        ],
        compiler_params=pltpu.CompilerParams(collective_id=0),
    )(x, w_mat)


# device time: 146740 ns/iter; 1.2519x vs baseline; 1.0053x over previous
import jax
import jax.numpy as jnp
from jax import lax
from jax.experimental import pallas as pl
from jax.experimental.pallas import tpu as pltpu

N_DEV = 16


def kernel(x, w_mat):
    x = x.astype(jnp.bfloat16)
    w_mat = w_mat.astype(jnp.bfloat16)
    m_per, k = x.shape
    _, n_per = w_mat.shape
    mh = m_per // 2

    def body(x_ref, w_ref, out_ref, comm_ref,
             ss_f, rs_f, ss_b, rs_b, ss_x, rs_x):
        my_pos = lax.axis_index("i")
        z = my_pos // 4
        i = lax.rem(my_pos, 4)
        xc = jnp.where((i == 1) | (i == 2), 1, 0)
        yc = jnp.where(i >= 2, 1, 0)
        i0 = xc
        i1 = 3 - xc
        j = jnp.where(yc == 0, z, 7 - z)

        def ring_pos(jj):
            jj = lax.rem(jj + 16, 8)
            return jnp.where(jj < 4, 4 * jj + i0, 4 * (7 - jj) + i1)

        def mate(q):
            return jnp.bitwise_xor(q, 1)

        fwd = ring_pos(j + 1)
        bwd = ring_pos(j - 1)
        partner = mate(my_pos)

        A = pl.ds(0, mh)
        B = pl.ds(mh, mh)

        barrier_sem = pltpu.get_barrier_semaphore()
        for nbr in [fwd, bwd, partner]:
            pl.semaphore_signal(
                barrier_sem, inc=1,
                device_id=(nbr,), device_id_type=pl.DeviceIdType.MESH,
            )
        comm_ref[0] = x_ref[...]
        pl.semaphore_wait(barrier_sem, 3)

        sends = []

        def start(src, dst, ss, rs, idx, target):
            src_ref = comm_ref.at[src] if isinstance(src, int) \
                else comm_ref.at[src[0], src[1]]
            dst_ref = comm_ref.at[dst] if isinstance(dst, int) \
                else comm_ref.at[dst[0], dst[1]]
            rd = pltpu.make_async_remote_copy(
                src_ref=src_ref, dst_ref=dst_ref,
                send_sem=ss.at[idx], recv_sem=rs.at[idx],
                device_id=(target,), device_id_type=pl.DeviceIdType.MESH,
            )
            rd.start()
            sends.append(rd)
            return rd

        def gemm(slot, origin):
            out_ref[pl.ds(origin * m_per, m_per), :] = jnp.dot(
                comm_ref[slot], w_ref[...],
                preferred_element_type=jnp.float32,
            )

        x0 = start(0, 1, ss_x, rs_x, 0, partner)
        f1 = start(0, 2, ss_f, rs_f, 0, fwd)
        b1 = start(0, 8, ss_b, rs_b, 0, bwd)
        gemm(0, my_pos)

        x0.wait_recv()
        gemm(1, partner)

        f1.wait_recv()
        f2 = start(2, 3, ss_f, rs_f, 1, fwd)
        x1 = start(2, 12, ss_x, rs_x, 1, partner)
        gemm(2, ring_pos(j - 1))

        b1.wait_recv()
        b2 = start(8, 9, ss_b, rs_b, 1, bwd)
        gemm(8, ring_pos(j + 1))

        f2.wait_recv()
        f3 = start(3, 4, ss_f, rs_f, 2, fwd)
        x3a = start((3, A), (14, A), ss_x, rs_x, 2, partner)
        x3b = start((3, B), (14, B), ss_x, rs_x, 3, partner)
        gemm(3, ring_pos(j - 2))

        b2.wait_recv()
        b3 = start(9, 10, ss_b, rs_b, 2, bwd)
        x4a = start((9, A), (15, A), ss_x, rs_x, 4, partner)
        x4b = start((9, B), (15, B), ss_x, rs_x, 5, partner)
        x2 = start(8, 13, ss_x, rs_x, 6, partner)
        gemm(9, ring_pos(j + 2))

        x3a.wait_recv()
        f4a = start((14, A), (5, A), ss_f, rs_f, 3, fwd)
        f3.wait_recv()
        f5 = start((4, A), (6, A), ss_f, rs_f, 5, fwd)
        gemm(4, ring_pos(j - 3))
        x3b.wait_recv()
        f4b = start((14, B), (5, B), ss_f, rs_f, 4, fwd)
        gemm(14, mate(ring_pos(j - 2)))

        b3.wait_recv()
        b5 = start((10, B), (6, B), ss_b, rs_b, 3, bwd)
        gemm(10, ring_pos(j + 3))

        x1.wait_recv()
        gemm(12, mate(ring_pos(j - 1)))

        x4a.wait_recv()
        b4a = start((15, A), (11, A), ss_b, rs_b, 4, bwd)
        x4b.wait_recv()
        b4b = start((15, B), (11, B), ss_b, rs_b, 5, bwd)
        gemm(15, mate(ring_pos(j + 2)))

        f4a.wait_recv()
        f6 = start((5, A), (7, A), ss_f, rs_f, 6, fwd)
        f4b.wait_recv()
        gemm(5, mate(ring_pos(j - 3)))

        b5.wait_recv()
        f5.wait_recv()
        q4 = ring_pos(j + 4)
        gemm(6, q4)

        x2.wait_recv()
        gemm(13, mate(ring_pos(j + 1)))

        b4a.wait_recv()
        b4b.wait_recv()
        b6 = start((11, B), (7, B), ss_b, rs_b, 6, bwd)
        gemm(11, mate(ring_pos(j + 3)))

        f6.wait_recv()
        b6.wait_recv()
        gemm(7, mate(q4))

        for rd in sends:
            rd.wait_send()

    return pl.pallas_call(
        body,
        out_shape=jax.ShapeDtypeStruct((N_DEV * m_per, n_per), jnp.float32),
        in_specs=[
            pl.BlockSpec(memory_space=pltpu.VMEM),
            pl.BlockSpec(memory_space=pltpu.VMEM),
        ],
        out_specs=pl.BlockSpec(memory_space=pltpu.VMEM),
        scratch_shapes=[
            pltpu.VMEM((N_DEV, m_per, k), jnp.bfloat16),
            pltpu.SemaphoreType.DMA((7,)),
            pltpu.SemaphoreType.DMA((7,)),
            pltpu.SemaphoreType.DMA((7,)),
            pltpu.SemaphoreType.DMA((7,)),
            pltpu.SemaphoreType.DMA((7,)),
            pltpu.SemaphoreType.DMA((7,)),
        ],
        compiler_params=pltpu.CompilerParams(collective_id=0),
    )(x, w_mat)
